# baseline (speedup 1.0000x reference)
import functools

import jax
import jax.numpy as jnp
from jax import lax
from jax.experimental import pallas as pl
from jax.experimental.pallas import tpu as pltpu
from jax.experimental.pallas import tpu_sc as plsc

NC = 2
NS = 16
CHUNK = 128
PIECE = 8
K_SPLIT = 128


def _round_up(a, b):
    return (a + b - 1) // b * b


def _sc_aggregate(x, col3, row3, zeros, n_pad):
    n_chunks = col3.shape[1]
    f = x.shape[1]
    rows_per_tile = n_pad // NS
    pieces0 = K_SPLIT // PIECE
    pieces1 = (n_chunks - K_SPLIT) // PIECE
    max_pieces = max(pieces0, pieces1)
    mesh = plsc.VectorSubcoreMesh(core_axis_name="c", subcore_axis_name="s")

    @functools.partial(
        pl.kernel,
        mesh=mesh,
        out_type=jax.ShapeDtypeStruct((NC, n_pad, f), jnp.float32),
        scratch_types=[
            pltpu.VMEM((PIECE, CHUNK), jnp.int32),
            pltpu.VMEM((PIECE, CHUNK), jnp.int32),
            pltpu.VMEM((2, CHUNK, f), jnp.float32),
            pltpu.VMEM_SHARED((n_pad, f), jnp.float32),
            pltpu.SemaphoreType.DMA,
            pltpu.SemaphoreType.DMA,
            pltpu.SemaphoreType.DMA,
            pltpu.SemaphoreType.DMA,
        ],
    )
    def agg(x_hbm, col_hbm, row_hbm, zero_hbm, out_hbm,
            col_v, row_v, rows_v, acc, gsem0, gsem1, ssem0, ssem1):
        c = lax.axis_index("c")
        s = lax.axis_index("s")
        gsems = (gsem0, gsem1)
        ssems = (ssem0, ssem1)
        tile_rows = pl.ds(s * rows_per_tile, rows_per_tile)
        base_piece = jnp.where(c == 0, 0, pieces0)
        n_pieces = jnp.where(c == 0, pieces0, pieces1)
        pltpu.sync_copy(zero_hbm.at[tile_rows], acc.at[tile_rows])
        plsc.subcore_barrier()

        def gather(j, b):
            pltpu.async_copy(x_hbm.at[col_v.at[j]], rows_v.at[b], gsems[b])

        def gather_wait(b):
            pltpu.make_async_copy(x_hbm.at[col_v.at[0]], rows_v.at[b],
                                  gsems[b]).wait()

        def scatter(j, b):
            pltpu.async_copy(rows_v.at[b], acc.at[row_v.at[j]], ssems[b],
                             add=True)

        def scatter_wait(b):
            pltpu.make_async_copy(rows_v.at[b], acc.at[pl.ds(0, CHUNK)],
                                  ssems[b]).wait()

        def run_piece(h):
            off = pl.ds((base_piece + h) * PIECE, PIECE)
            pltpu.sync_copy(col_hbm.at[s, off], col_v)
            pltpu.sync_copy(row_hbm.at[s, off], row_v)
            gather(0, 0)
            gather(1, 1)

            def body(i, carry):
                for b in range(2):
                    gather_wait(b)
                    scatter(2 * i + b, b)
                for b in range(2):
                    scatter_wait(b)

                    @pl.when(2 * i + b + 2 < PIECE)
                    def _():
                        gather(2 * i + b + 2, b)

                return carry

            lax.fori_loop(0, PIECE // 2, body, 0)

        for h in range(max_pieces):
            @pl.when(h < n_pieces)
            def _():
                run_piece(h)

        plsc.subcore_barrier()
        pltpu.sync_copy(acc.at[tile_rows], out_hbm.at[c, tile_rows])

    return agg(x, col3, row3, zeros)


def _tc_matmul_bias(parts, weight, bias):
    n_pad, f = parts.shape[1], parts.shape[2]
    blk = next(r for r in (1024, 512, 256, 128, 8) if n_pad % r == 0)

    def body(p_ref, w_ref, b_ref, o_ref):
        psum = p_ref[0] + p_ref[1]
        o_ref[...] = (
            jnp.dot(psum, w_ref[...], preferred_element_type=jnp.float32)
            + b_ref[...]
        )

    return pl.pallas_call(
        body,
        grid=(n_pad // blk,),
        in_specs=[
            pl.BlockSpec((2, blk, f), lambda i: (0, i, 0)),
            pl.BlockSpec((f, f), lambda i: (0, 0)),
            pl.BlockSpec((1, f), lambda i: (0, 0)),
        ],
        out_specs=pl.BlockSpec((blk, f), lambda i: (i, 0)),
        out_shape=jax.ShapeDtypeStruct((n_pad, f), jnp.float32),
    )(parts, weight, bias.reshape(1, f))


def kernel(x, edge_index, weight, bias):
    n_nodes, f = x.shape
    e = edge_index.shape[1]
    ei = edge_index.astype(jnp.int32)
    row, col = ei[0], ei[1]

    n_pad = _round_up(n_nodes + 1, 128)
    dummy_row = n_nodes

    n_chunks = _round_up((e + NS - 1) // NS, 2 * PIECE * CHUNK) // CHUNK
    e_pad = NS * n_chunks * CHUNK
    col_p = jnp.zeros((e_pad,), jnp.int32).at[:e].set(col)
    row_p = jnp.full((e_pad,), dummy_row, jnp.int32).at[:e].set(row)
    col3 = col_p.reshape(NS, n_chunks, CHUNK)
    row3 = row_p.reshape(NS, n_chunks, CHUNK)
    zeros = jnp.zeros((n_pad, f), jnp.float32)

    parts = _sc_aggregate(x, col3, row3, zeros, n_pad)
    out = _tc_matmul_bias(parts, weight, bias)
    return out[:n_nodes]


# speedup vs baseline: 1.1127x; 1.1127x over previous
import functools

import jax
import jax.numpy as jnp
from jax import lax
from jax.experimental import pallas as pl
from jax.experimental.pallas import tpu as pltpu
from jax.experimental.pallas import tpu_sc as plsc

NC = 2
NS = 16
CHUNK = 128
PIECE = 40
K_SPLIT = 120


def _round_up(a, b):
    return (a + b - 1) // b * b


def _sc_aggregate(x, col3, row3, zeros, n_pad):
    n_chunks = col3.shape[1]
    f = x.shape[1]
    rows_per_tile = n_pad // NS
    pieces0 = K_SPLIT // PIECE
    pieces1 = (n_chunks - K_SPLIT) // PIECE
    max_pieces = max(pieces0, pieces1)
    mesh = plsc.VectorSubcoreMesh(core_axis_name="c", subcore_axis_name="s")

    @functools.partial(
        pl.kernel,
        mesh=mesh,
        out_type=jax.ShapeDtypeStruct((NC, n_pad, f), jnp.float32),
        scratch_types=[
            pltpu.VMEM((PIECE, CHUNK), jnp.int32),
            pltpu.VMEM((PIECE, CHUNK), jnp.int32),
            pltpu.VMEM((2, CHUNK, f), jnp.float32),
            pltpu.VMEM_SHARED((n_pad, f), jnp.float32),
            pltpu.SemaphoreType.DMA,
            pltpu.SemaphoreType.DMA,
            pltpu.SemaphoreType.DMA,
            pltpu.SemaphoreType.DMA,
        ],
    )
    def agg(x_hbm, col_hbm, row_hbm, zero_hbm, out_hbm,
            col_v, row_v, rows_v, acc, gsem0, gsem1, ssem0, ssem1):
        c = lax.axis_index("c")
        s = lax.axis_index("s")
        gsems = (gsem0, gsem1)
        ssems = (ssem0, ssem1)
        tile_rows = pl.ds(s * rows_per_tile, rows_per_tile)
        base_piece = jnp.where(c == 0, 0, pieces0)
        n_pieces = jnp.where(c == 0, pieces0, pieces1)
        pltpu.sync_copy(zero_hbm.at[tile_rows], acc.at[tile_rows])
        plsc.subcore_barrier()

        def gather(j, b):
            pltpu.async_copy(x_hbm.at[col_v.at[j]], rows_v.at[b], gsems[b])

        def gather_wait(b):
            pltpu.make_async_copy(x_hbm.at[col_v.at[0]], rows_v.at[b],
                                  gsems[b]).wait()

        def scatter(j, b):
            pltpu.async_copy(rows_v.at[b], acc.at[row_v.at[j]], ssems[b],
                             add=True)

        def scatter_wait(b):
            pltpu.make_async_copy(rows_v.at[b], acc.at[pl.ds(0, CHUNK)],
                                  ssems[b]).wait()

        def run_piece(h):
            off = pl.ds((base_piece + h) * PIECE, PIECE)
            pltpu.sync_copy(col_hbm.at[s, off], col_v)
            pltpu.sync_copy(row_hbm.at[s, off], row_v)
            gather(0, 0)
            gather(1, 1)

            def body(i, carry):
                for b in range(2):
                    gather_wait(b)
                    scatter(2 * i + b, b)
                for b in range(2):
                    scatter_wait(b)

                    @pl.when(2 * i + b + 2 < PIECE)
                    def _():
                        gather(2 * i + b + 2, b)

                return carry

            lax.fori_loop(0, PIECE // 2, body, 0)

        for h in range(max_pieces):
            @pl.when(h < n_pieces)
            def _():
                run_piece(h)

        plsc.subcore_barrier()
        pltpu.sync_copy(acc.at[tile_rows], out_hbm.at[c, tile_rows])

    return agg(x, col3, row3, zeros)


def _tc_matmul_bias(parts, weight, bias):
    n_pad, f = parts.shape[1], parts.shape[2]
    blk = next(r for r in (1024, 512, 256, 128, 8) if n_pad % r == 0)

    def body(p_ref, w_ref, b_ref, o_ref):
        psum = p_ref[0] + p_ref[1]
        o_ref[...] = (
            jnp.dot(psum, w_ref[...], preferred_element_type=jnp.float32)
            + b_ref[...]
        )

    return pl.pallas_call(
        body,
        grid=(n_pad // blk,),
        in_specs=[
            pl.BlockSpec((2, blk, f), lambda i: (0, i, 0)),
            pl.BlockSpec((f, f), lambda i: (0, 0)),
            pl.BlockSpec((1, f), lambda i: (0, 0)),
        ],
        out_specs=pl.BlockSpec((blk, f), lambda i: (i, 0)),
        out_shape=jax.ShapeDtypeStruct((n_pad, f), jnp.float32),
    )(parts, weight, bias.reshape(1, f))


def kernel(x, edge_index, weight, bias):
    n_nodes, f = x.shape
    e = edge_index.shape[1]
    ei = edge_index.astype(jnp.int32)
    row, col = ei[0], ei[1]

    n_pad = _round_up(n_nodes + 1, 128)
    dummy_row = n_nodes

    n_chunks = _round_up((e + NS - 1) // NS, 2 * PIECE * CHUNK) // CHUNK
    e_pad = NS * n_chunks * CHUNK
    col_p = jnp.zeros((e_pad,), jnp.int32).at[:e].set(col)
    row_p = jnp.full((e_pad,), dummy_row, jnp.int32).at[:e].set(row)
    col3 = col_p.reshape(NS, n_chunks, CHUNK)
    row3 = row_p.reshape(NS, n_chunks, CHUNK)
    zeros = jnp.zeros((n_pad, f), jnp.float32)

    parts = _sc_aggregate(x, col3, row3, zeros, n_pad)
    out = _tc_matmul_bias(parts, weight, bias)
    return out[:n_nodes]


# speedup vs baseline: 1.2033x; 1.0815x over previous
import functools

import jax
import jax.numpy as jnp
from jax import lax
from jax.experimental import pallas as pl
from jax.experimental.pallas import tpu as pltpu
from jax.experimental.pallas import tpu_sc as plsc

NC = 2
NS = 16
CHUNK = 128
PIECE = 40
K_SPLIT = 120


def _round_up(a, b):
    return (a + b - 1) // b * b


def _sc_aggregate(x, col3, row3, zeros, n_pad):
    n_chunks = col3.shape[1]
    f = x.shape[1]
    rows_per_tile = n_pad // NS
    pieces0 = K_SPLIT // PIECE
    pieces1 = (n_chunks - K_SPLIT) // PIECE
    max_pieces = max(pieces0, pieces1)
    mesh = plsc.VectorSubcoreMesh(core_axis_name="c", subcore_axis_name="s")

    @functools.partial(
        pl.kernel,
        mesh=mesh,
        out_type=jax.ShapeDtypeStruct((NC, n_pad, f), jnp.float32),
        scratch_types=[
            pltpu.VMEM((PIECE, CHUNK), jnp.int32),
            pltpu.VMEM((PIECE, CHUNK), jnp.int32),
            pltpu.VMEM((2, CHUNK, f), jnp.float32),
            pltpu.VMEM_SHARED((n_pad, f), jnp.float32),
            pltpu.SemaphoreType.DMA,
            pltpu.SemaphoreType.DMA,
            pltpu.SemaphoreType.DMA,
            pltpu.SemaphoreType.DMA,
        ],
    )
    def agg(x_hbm, col_hbm, row_hbm, zero_hbm, out_hbm,
            col_v, row_v, rows_v, acc, gsem0, gsem1, ssem0, ssem1):
        c = lax.axis_index("c")
        s = lax.axis_index("s")
        gsems = (gsem0, gsem1)
        ssems = (ssem0, ssem1)
        tile_rows = pl.ds(s * rows_per_tile, rows_per_tile)
        base_piece = jnp.where(c == 0, 0, pieces0)
        n_pieces = jnp.where(c == 0, pieces0, pieces1)
        pltpu.sync_copy(zero_hbm.at[tile_rows], acc.at[tile_rows])
        plsc.subcore_barrier()

        def gather(j, b):
            pltpu.async_copy(x_hbm.at[col_v.at[j]], rows_v.at[b], gsems[b])

        def gather_wait(b):
            pltpu.make_async_copy(x_hbm.at[col_v.at[0]], rows_v.at[b],
                                  gsems[b]).wait()

        def scatter(j, b):
            pltpu.async_copy(rows_v.at[b], acc.at[row_v.at[j]], ssems[b],
                             add=True)

        def scatter_wait(b):
            pltpu.make_async_copy(rows_v.at[b], acc.at[pl.ds(0, CHUNK)],
                                  ssems[b]).wait()

        def run_piece(h):
            off = pl.ds((base_piece + h) * PIECE, PIECE)
            pltpu.sync_copy(col_hbm.at[s, off], col_v)
            pltpu.sync_copy(row_hbm.at[s, off], row_v)
            gather(0, 0)
            gather(1, 1)

            def body(i, carry):
                for b in range(2):
                    gather_wait(b)
                    scatter(2 * i + b, b)
                for b in range(2):
                    scatter_wait(b)

                    @pl.when(2 * i + b + 2 < PIECE)
                    def _():
                        gather(2 * i + b + 2, b)

                return carry

            lax.fori_loop(0, PIECE // 2, body, 0)

        for h in range(max_pieces):
            @pl.when(h < n_pieces)
            def _():
                run_piece(h)

        plsc.subcore_barrier()
        pltpu.sync_copy(acc.at[tile_rows], out_hbm.at[c, tile_rows])

    return agg(x, col3, row3, zeros)


def _tc_matmul_bias(parts, weight, bias):
    n_pad, f = parts.shape[1], parts.shape[2]
    blk = next(r for r in (1024, 512, 256, 128, 8) if n_pad % r == 0)

    def body(p_ref, w_ref, b_ref, o_ref):
        psum = p_ref[0] + p_ref[1]
        o_ref[...] = (
            jnp.dot(psum, w_ref[...], preferred_element_type=jnp.float32)
            + b_ref[...]
        )

    return pl.pallas_call(
        body,
        grid=(n_pad // blk,),
        in_specs=[
            pl.BlockSpec((2, blk, f), lambda i: (0, i, 0)),
            pl.BlockSpec((f, f), lambda i: (0, 0)),
            pl.BlockSpec((1, f), lambda i: (0, 0)),
        ],
        out_specs=pl.BlockSpec((blk, f), lambda i: (i, 0)),
        out_shape=jax.ShapeDtypeStruct((n_pad, f), jnp.float32),
    )(parts, weight, bias.reshape(1, f))


def kernel(x, edge_index, weight, bias):
    n_nodes, f = x.shape
    e = edge_index.shape[1]
    ei = edge_index.astype(jnp.int32)
    row, col = ei[0], ei[1]

    n_pad = _round_up(n_nodes + 1, 1024)
    dummy_row = n_nodes

    n_chunks = _round_up((e + NS - 1) // NS, 2 * PIECE * CHUNK) // CHUNK
    e_pad = NS * n_chunks * CHUNK
    col_p = jnp.zeros((e_pad,), jnp.int32).at[:e].set(col)
    row_p = jnp.full((e_pad,), dummy_row, jnp.int32).at[:e].set(row)
    col3 = col_p.reshape(NS, n_chunks, CHUNK)
    row3 = row_p.reshape(NS, n_chunks, CHUNK)
    zeros = jnp.zeros((n_pad, f), jnp.float32)

    parts = _sc_aggregate(x, col3, row3, zeros, n_pad)
    out = _tc_matmul_bias(parts, weight, bias)
    return out[:n_nodes]
